# baseline (speedup 1.0000x reference)
import functools

import jax
import jax.numpy as jnp
from jax import lax
from jax.experimental import pallas as pl
from jax.experimental.pallas import tpu as pltpu
from jax.experimental.pallas import tpu_sc as plsc

_NUM_FIELDS = 26
_FIELD_SIZE = 100000
_BATCH = 16384
_NC = 2
_NS = 16
_NW = _NC * _NS
_BPW = _BATCH // _NW
_LANES = 16
_IPW = _NUM_FIELDS * _BPW
_VPF = _BPW // _LANES


_PAD_ROWS = 2600960
_CPW = _PAD_ROWS // _NW
_CPW_LAST = 2600000 - (_NW - 1) * _CPW


def _sc_relayout(table):
    mesh = plsc.VectorSubcoreMesh(core_axis_name="c", subcore_axis_name="s")

    @functools.partial(
        pl.kernel,
        out_type=jax.ShapeDtypeStruct((_PAD_ROWS, 1), jnp.float32),
        mesh=mesh,
    )
    def k(table_hbm, out_hbm):
        wid = lax.axis_index("s") * _NC + lax.axis_index("c")
        start = wid * _CPW

        def full(_):
            pltpu.sync_copy(
                table_hbm.at[pl.ds(start, _CPW), :],
                out_hbm.at[pl.ds(start, _CPW), :],
            )
            return 0

        def tail(_):
            tstart = (_NW - 1) * _CPW
            pltpu.sync_copy(
                table_hbm.at[pl.ds(tstart, _CPW_LAST), :],
                out_hbm.at[pl.ds(tstart, _CPW_LAST), :],
            )
            return 0

        lax.cond(wid < _NW - 1, full, tail, 0)

    return k(table)


def _sc_embed_sum(xw, table_flat):
    mesh = plsc.VectorSubcoreMesh(core_axis_name="c", subcore_axis_name="s")

    @functools.partial(
        pl.kernel,
        out_type=jax.ShapeDtypeStruct((_BATCH,), jnp.float32),
        mesh=mesh,
        cost_estimate=pl.CostEstimate(
            flops=0, transcendentals=0, bytes_accessed=1024
        ),
        scratch_types=[
            pltpu.VMEM((_IPW,), jnp.int32),
            pltpu.VMEM((_IPW,), jnp.float32),
            pltpu.VMEM((_BPW,), jnp.float32),
            pltpu.SemaphoreType.DMA,
            pltpu.SemaphoreType.DMA,
        ],
    )
    def k(xw_hbm, table_hbm, out_hbm, idx_v, vals_v, out_v, sem, sem2):
        wid = lax.axis_index("s") * _NC + lax.axis_index("c")
        pltpu.sync_copy(xw_hbm.at[wid], idx_v)

        _H = _IPW // 2
        c1 = pltpu.async_copy(
            table_hbm.at[idx_v.at[pl.ds(0, _H)]],
            vals_v.at[pl.ds(0, _H)],
            sem,
        )
        c2 = pltpu.async_copy(
            table_hbm.at[idx_v.at[pl.ds(_H, _H)]],
            vals_v.at[pl.ds(_H, _H)],
            sem2,
        )
        c1.wait()
        c2.wait()

        for v in range(_VPF):
            base = v * _LANES

            def body(f, acc):
                return acc + vals_v[pl.ds(f * _BPW + base, _LANES)]

            acc = lax.fori_loop(
                0, _NUM_FIELDS, body, jnp.zeros((_LANES,), jnp.float32)
            )
            out_v[pl.ds(base, _LANES)] = acc

        pltpu.sync_copy(out_v, out_hbm.at[pl.ds(wid * _BPW, _BPW)])

    return k(xw, table_flat)


def kernel(x, table):
    x = x.astype(jnp.int32)
    offsets = jnp.arange(_NUM_FIELDS, dtype=jnp.int32) * _FIELD_SIZE
    xw = (
        (x + offsets[None, :])
        .reshape(_NW, _BPW, _NUM_FIELDS)
        .transpose(0, 2, 1)
        .reshape(_NW, _IPW)
    )
    table_flat = _sc_relayout(table).reshape(-1)
    out = _sc_embed_sum(xw, table_flat)
    return out.reshape(_BATCH, 1)


# speedup vs baseline: 722.3045x; 722.3045x over previous
import functools

import jax
import jax.numpy as jnp
from jax import lax
from jax.experimental import pallas as pl
from jax.experimental.pallas import tpu as pltpu
from jax.experimental.pallas import tpu_sc as plsc

_NUM_FIELDS = 26
_FIELD_SIZE = 100000
_BATCH = 16384
_NC = 2
_NS = 16
_NW = _NC * _NS
_BPW = _BATCH // _NW
_LANES = 16
_IPW = _NUM_FIELDS * _BPW
_VPF = _BPW // _LANES


def _sc_embed_sum(xw, table_flat):
    mesh = plsc.VectorSubcoreMesh(core_axis_name="c", subcore_axis_name="s")

    @functools.partial(
        pl.kernel,
        out_type=jax.ShapeDtypeStruct((_BATCH,), jnp.float32),
        mesh=mesh,
        cost_estimate=pl.CostEstimate(
            flops=0, transcendentals=0, bytes_accessed=1024
        ),
        scratch_types=[
            pltpu.VMEM((_IPW,), jnp.int32),
            pltpu.VMEM((_IPW,), jnp.float32),
            pltpu.VMEM((_BPW,), jnp.float32),
            pltpu.SemaphoreType.DMA,
            pltpu.SemaphoreType.DMA,
        ],
    )
    def k(xw_hbm, table_hbm, out_hbm, idx_v, vals_v, out_v, sem, sem2):
        wid = lax.axis_index("s") * _NC + lax.axis_index("c")
        pltpu.sync_copy(xw_hbm.at[wid], idx_v)

        _H = _IPW // 2
        c1 = pltpu.async_copy(
            table_hbm.at[idx_v.at[pl.ds(0, _H)]],
            vals_v.at[pl.ds(0, _H)],
            sem,
        )
        c2 = pltpu.async_copy(
            table_hbm.at[idx_v.at[pl.ds(_H, _H)]],
            vals_v.at[pl.ds(_H, _H)],
            sem2,
        )
        _FH = _NUM_FIELDS // 2
        c1.wait()
        for v in range(_VPF):
            base = v * _LANES
            acc = vals_v[pl.ds(base, _LANES)]
            for f in range(1, _FH):
                acc = acc + vals_v[pl.ds(f * _BPW + base, _LANES)]
            out_v[pl.ds(base, _LANES)] = acc

        c2.wait()
        for v in range(_VPF):
            base = v * _LANES
            acc = out_v[pl.ds(base, _LANES)]
            for f in range(_FH, _NUM_FIELDS):
                acc = acc + vals_v[pl.ds(f * _BPW + base, _LANES)]
            out_v[pl.ds(base, _LANES)] = acc

        pltpu.sync_copy(out_v, out_hbm.at[pl.ds(wid * _BPW, _BPW)])

    return k(xw, table_flat)


def kernel(x, table):
    x = x.astype(jnp.int32)
    offsets = jnp.arange(_NUM_FIELDS, dtype=jnp.int32) * _FIELD_SIZE
    xw = (
        (x + offsets[None, :])
        .reshape(_NW, _BPW, _NUM_FIELDS)
        .transpose(0, 2, 1)
        .reshape(_NW, _IPW)
    )
    table_flat = jnp.pad(table, ((0, 960), (0, 0))).reshape(-1)
    out = _sc_embed_sum(xw, table_flat)
    return out.reshape(_BATCH, 1)


# speedup vs baseline: 1018.4485x; 1.4100x over previous
import functools

import jax
import jax.numpy as jnp
from jax import lax
from jax.experimental import pallas as pl
from jax.experimental.pallas import tpu as pltpu
from jax.experimental.pallas import tpu_sc as plsc

_NUM_FIELDS = 26
_FIELD_SIZE = 100000
_BATCH = 16384
_NC = 2
_NS = 16
_NW = _NC * _NS
_BPW = _BATCH // _NW
_LANES = 16
_IPW = _NUM_FIELDS * _BPW
_VPF = _BPW // _LANES


def _sc_embed_sum(xw, table_flat):
    mesh = plsc.VectorSubcoreMesh(core_axis_name="c", subcore_axis_name="s")

    @functools.partial(
        pl.kernel,
        out_type=jax.ShapeDtypeStruct((_BATCH,), jnp.float32),
        mesh=mesh,
        cost_estimate=pl.CostEstimate(
            flops=0, transcendentals=0, bytes_accessed=1024
        ),
        scratch_types=[
            pltpu.VMEM((_IPW,), jnp.int32),
            pltpu.VMEM((_IPW,), jnp.float32),
            pltpu.VMEM((_BPW,), jnp.float32),
            pltpu.SemaphoreType.DMA,
            pltpu.SemaphoreType.DMA,
        ],
    )
    def k(xw_hbm, table_hbm, out_hbm, idx_v, vals_v, out_v, sem, sem2):
        wid = lax.axis_index("s") * _NC + lax.axis_index("c")
        pltpu.sync_copy(xw_hbm.at[wid], idx_v)

        _H = _IPW // 2
        c1 = pltpu.async_copy(
            table_hbm.at[0].at[idx_v.at[pl.ds(0, _H)]],
            vals_v.at[pl.ds(0, _H)],
            sem,
        )
        c2 = pltpu.async_copy(
            table_hbm.at[0].at[idx_v.at[pl.ds(_H, _H)]],
            vals_v.at[pl.ds(_H, _H)],
            sem2,
        )
        _FH = _NUM_FIELDS // 2
        c1.wait()
        for v in range(_VPF):
            base = v * _LANES
            acc = vals_v[pl.ds(base, _LANES)]
            for f in range(1, _FH):
                acc = acc + vals_v[pl.ds(f * _BPW + base, _LANES)]
            out_v[pl.ds(base, _LANES)] = acc

        c2.wait()
        for v in range(_VPF):
            base = v * _LANES
            acc = out_v[pl.ds(base, _LANES)]
            for f in range(_FH, _NUM_FIELDS):
                acc = acc + vals_v[pl.ds(f * _BPW + base, _LANES)]
            out_v[pl.ds(base, _LANES)] = acc

        pltpu.sync_copy(out_v, out_hbm.at[pl.ds(wid * _BPW, _BPW)])

    return k(xw, table_flat)


def kernel(x, table):
    x = x.astype(jnp.int32)
    offsets = jnp.arange(_NUM_FIELDS, dtype=jnp.int32) * _FIELD_SIZE
    xw = (
        (x + offsets[None, :])
        .reshape(_NW, _BPW, _NUM_FIELDS)
        .transpose(0, 2, 1)
        .reshape(_NW, _IPW)
    )
    out = _sc_embed_sum(xw, table.reshape(1, -1))
    return out.reshape(_BATCH, 1)
